# baseline (speedup 1.0000x reference)
import functools

import jax
import jax.numpy as jnp
from jax import lax
from jax.experimental import pallas as pl
from jax.experimental.pallas import tpu as pltpu
from jax.experimental.pallas import tpu_sc as plsc

_VOCAB = 1_000_000
_BATCH, _HIST, _D = 16384, 200, 16
_TOTAL = _BATCH * _HIST
_NC, _NS = 2, 16
_NW = _NC * _NS

_VPAD = 1_000_064
_TR = 1664
_TLINES = _TR * _D // 128
_TCHUNKS = _VPAD // _TR

_PER_W = _TOTAL // _NW
_CHUNK = 2048
_NBUF = 2
_NCHUNK = _PER_W // _CHUNK
_NOUTER = _NCHUNK // _NBUF


def _make_transpose():
    mesh = plsc.VectorSubcoreMesh(core_axis_name="c", subcore_axis_name="s")

    @functools.partial(
        pl.kernel,
        mesh=mesh,
        compiler_params=pltpu.CompilerParams(needs_layout_passes=False),
        out_type=jax.ShapeDtypeStruct((_VPAD * _D // 128, 128), jnp.float32),
        scratch_types=[
            pltpu.VMEM((_D, _TR + 1), jnp.float32),
            pltpu.VMEM((_TLINES, 128), jnp.float32),
        ],
    )
    def t(embt, tbl, xv, yv):
        wid = lax.axis_index("s") * _NC + lax.axis_index("c")
        lanes = lax.iota(jnp.int32, 16)

        def body(k, carry):
            ch = wid + _NW * k

            @pl.when(ch < _TCHUNKS)
            def _():
                pltpu.sync_copy(
                    embt.at[:, pl.ds(ch * _TR, _TR)],
                    xv.at[:, pl.ds(0, _TR)])

                def row(r, carry2):
                    v = plsc.load_gather(xv, [lanes, jnp.full((16,), r, jnp.int32)])
                    yv[r >> 3, pl.ds((r & 7) * _D, _D)] = v
                    return carry2

                lax.fori_loop(0, _TR, row, 0)
                pltpu.sync_copy(yv, tbl.at[pl.ds(ch * _TLINES, _TLINES), :])
            return carry

        lax.fori_loop(0, (_TCHUNKS + _NW - 1) // _NW, body, 0)

    return t


def _make_gather():
    mesh = plsc.VectorSubcoreMesh(core_axis_name="c", subcore_axis_name="s")

    @functools.partial(
        pl.kernel,
        mesh=mesh,
        compiler_params=pltpu.CompilerParams(use_tc_tiling_on_sc=False),
        out_type=jax.ShapeDtypeStruct((_TOTAL, _D), jnp.float32),
        scratch_types=[
            pltpu.VMEM((_NBUF, _CHUNK), jnp.int32),
            pltpu.VMEM((_NBUF, _CHUNK, _D), jnp.float32),
            pltpu.SemaphoreType.DMA,
            pltpu.SemaphoreType.DMA,
            pltpu.SemaphoreType.DMA,
            pltpu.SemaphoreType.DMA,
            pltpu.SemaphoreType.DMA,
            pltpu.SemaphoreType.DMA,
        ],
    )
    def k(table, idx, out, idx_v, rows_v, i0, i1, g0, g1, o0, o1):
        isem = (i0, i1)
        gsem = (g0, g1)
        osem = (o0, o1)
        wid = lax.axis_index("s") * _NC + lax.axis_index("c")
        base = wid * _PER_W

        def idx_copy(c, b):
            return pltpu.make_async_copy(
                idx.at[pl.ds(base + c * _CHUNK, _CHUNK)], idx_v.at[b], isem[b])

        def gat_copy(b):
            return pltpu.make_async_copy(
                table.at[idx_v.at[b]], rows_v.at[b], gsem[b])

        def out_copy(c, b):
            return pltpu.make_async_copy(
                rows_v.at[b], out.at[pl.ds(base + c * _CHUNK, _CHUNK)], osem[b])

        for b in range(_NBUF):
            idx_copy(b, b).start()

        def body(go, carry):
            for b in range(_NBUF):
                c = go * _NBUF + b
                idx_copy(c, b).wait()

                @pl.when(go > 0)
                def _():
                    out_copy(c, b).wait()

                gat_copy(b).start()
                gat_copy(b).wait()

                @pl.when(go < _NOUTER - 1)
                def _():
                    idx_copy(c + _NBUF, b).start()

                out_copy(c, b).start()
            return carry

        lax.fori_loop(0, _NOUTER, body, 0)
        for b in range(_NBUF):
            out_copy(_NCHUNK - _NBUF + b, b).wait()

    return k


_transpose = _make_transpose()
_gather = _make_gather()


def kernel(x, emb_weight):
    idx = x.reshape(-1).astype(jnp.int32)
    embt = jnp.pad(emb_weight.T, ((0, 0), (0, _VPAD - _VOCAB)))
    tbl128 = _transpose(embt)
    tbl = tbl128.reshape(_VPAD, _D)
    out = _gather(tbl, idx)
    return out.reshape(_BATCH, _HIST, _D)


# speedup vs baseline: 1.0168x; 1.0168x over previous
import functools

import jax
import jax.numpy as jnp
from jax import lax
from jax.experimental import pallas as pl
from jax.experimental.pallas import tpu as pltpu
from jax.experimental.pallas import tpu_sc as plsc

_VOCAB = 1_000_000
_BATCH, _HIST, _D = 16384, 200, 16
_TOTAL = _BATCH * _HIST
_NC, _NS = 2, 16
_NW = _NC * _NS

_GB = 16
_CHUNK = _GB * _HIST
_WB = _BATCH // _NW
_NCHUNK = _WB // _GB
_NBUF = 2
_NOUTER = _NCHUNK // _NBUF


def _make_gather():
    mesh = plsc.VectorSubcoreMesh(core_axis_name="c", subcore_axis_name="s")

    @functools.partial(
        pl.kernel,
        mesh=mesh,
        compiler_params=pltpu.CompilerParams(use_tc_tiling_on_sc=False),
        out_type=jax.ShapeDtypeStruct((_BATCH, _HIST, _D), jnp.float32),
        scratch_types=[
            pltpu.VMEM((_NBUF, _GB, _HIST), jnp.int32),
            pltpu.VMEM((_NBUF, _GB, _HIST, _D), jnp.float32),
            pltpu.SemaphoreType.DMA,
            pltpu.SemaphoreType.DMA,
            pltpu.SemaphoreType.DMA,
            pltpu.SemaphoreType.DMA,
            pltpu.SemaphoreType.DMA,
            pltpu.SemaphoreType.DMA,
        ],
    )
    def k(idx, table, out, idx_v, rows_v, i0, i1, g0, g1, o0, o1):
        isem = (i0, i1)
        gsem = (g0, g1)
        osem = (o0, o1)
        wid = lax.axis_index("s") * _NC + lax.axis_index("c")
        brow = wid * _WB

        def idx_copy(c, b):
            return pltpu.make_async_copy(
                idx.at[pl.ds(brow + c * _GB, _GB), :], idx_v.at[b], isem[b])

        def gat_copy(b, g):
            return pltpu.make_async_copy(
                table.at[idx_v.at[b, g]], rows_v.at[b, g], gsem[b])

        def out_copy(c, b):
            return pltpu.make_async_copy(
                rows_v.at[b], out.at[pl.ds(brow + c * _GB, _GB)], osem[b])

        for b in range(_NBUF):
            idx_copy(b, b).start()

        def body(go, carry):
            for b in range(_NBUF):
                c = go * _NBUF + b
                idx_copy(c, b).wait()

                @pl.when(go > 0)
                def _():
                    out_copy(c, b).wait()

                for g in range(_GB):
                    gat_copy(b, g).start()
                for g in range(_GB):
                    gat_copy(b, g).wait()

                @pl.when(go < _NOUTER - 1)
                def _():
                    idx_copy(c + _NBUF, b).start()

                out_copy(c, b).start()
            return carry

        lax.fori_loop(0, _NOUTER, body, 0)
        for b in range(_NBUF):
            out_copy(_NCHUNK - _NBUF + b, b).wait()

    return k


_gather = _make_gather()


def kernel(x, emb_weight):
    return _gather(x.astype(jnp.int32), emb_weight)


# speedup vs baseline: 2.2597x; 2.2223x over previous
import functools

import jax
import jax.numpy as jnp
from jax import lax
from jax.experimental import pallas as pl
from jax.experimental.pallas import tpu as pltpu
from jax.experimental.pallas import tpu_sc as plsc

_VOCAB = 1_000_000
_BATCH, _HIST, _D = 16384, 200, 16
_NC, _NS = 2, 16
_NW = _NC * _NS

_GB = 8
_WB = _BATCH // _NW
_NCH = _WB // _GB
_PITCH = 9
_HH = _HIST // 2


def _make_gather():
    mesh = plsc.VectorSubcoreMesh(core_axis_name="c", subcore_axis_name="s")

    @functools.partial(
        pl.kernel,
        mesh=mesh,
        compiler_params=pltpu.CompilerParams(use_tc_tiling_on_sc=False,
                                            needs_layout_passes=False),
        out_type=jax.ShapeDtypeStruct((_HIST, 2, 128, 8, 128), jnp.float32),
        scratch_types=[
            pltpu.VMEM((_GB, _HIST), jnp.int32),
            pltpu.VMEM((_GB, _HIST), jnp.int32),
            pltpu.VMEM((_GB, _HIST, _D), jnp.float32),
            pltpu.VMEM((_GB, _HIST, _D), jnp.float32),
            pltpu.VMEM((_HH, 2, 1, 8, _PITCH), jnp.float32),
            pltpu.VMEM((_HH, 2, 1, 8, _PITCH), jnp.float32),
            pltpu.SemaphoreType.DMA,
            pltpu.SemaphoreType.DMA,
            pltpu.SemaphoreType.DMA,
            pltpu.SemaphoreType.DMA,
            pltpu.SemaphoreType.DMA,
            pltpu.SemaphoreType.DMA,
        ],
    )
    def k(x, table, out, xv0, xv1, rv0, rv1, yv0, yv1,
          i0, i1, g0, g1, o0, o1):
        xv = (xv0, xv1)
        rv = (rv0, rv1)
        yv = (yv0, yv1)
        isem = (i0, i1)
        gsem = (g0, g1)
        osem = (o0, o1)
        wid = lax.axis_index("s") * _NC + lax.axis_index("c")
        wb0 = wid * _WB

        lanes = lax.iota(jnp.int32, 16)
        td_i = lanes >> 3
        tb_i = jnp.zeros((16,), jnp.int32)
        sub_i = lanes & 7

        def idx_copy(c, s):
            return pltpu.make_async_copy(
                x.at[pl.ds(wb0 + c * _GB, _GB), :], xv[s], isem[s])

        def gat_copy(s, g):
            return pltpu.make_async_copy(
                table.at[xv[s].at[g]], rv[s].at[g], gsem[s])

        def out_copy(c, half):
            tb = (wb0 + c * _GB) >> 7
            l0 = pl.multiple_of((wb0 + c * _GB) & 127, _GB)
            return pltpu.make_async_copy(
                yv[half].at[:, :, :, :, pl.ds(0, _GB)],
                out.at[pl.ds(half * _HH, _HH), :, pl.ds(tb, 1), :,
                       pl.ds(l0, _GB)], osem[half])

        def transpose(s, half):
            def hrow(h, carry):
                h_i = jnp.full((16,), h, jnp.int32)
                for g in range(_GB):
                    v = rv[s][g, half * _HH + h, :]
                    plsc.store_scatter(
                        yv[half], [h_i, td_i, tb_i, sub_i,
                                   jnp.full((16,), g, jnp.int32)], v)
                return carry

            lax.fori_loop(0, _HH, hrow, 0)

        idx_copy(0, 0).start()
        idx_copy(1, 1).start()
        idx_copy(0, 0).wait()
        for g in range(_GB):
            gat_copy(0, g).start()

        def body(go, carry):
            for s in range(2):
                c = go * 2 + s
                ns = 1 - s

                @pl.when(c + 1 < _NCH)
                def _():
                    idx_copy(c + 1, ns).wait()
                    for g in range(_GB):
                        gat_copy(ns, g).start()

                for g in range(_GB):
                    gat_copy(s, g).wait()

                @pl.when(c + 2 < _NCH)
                def _():
                    idx_copy(c + 2, s).start()

                for half in range(2):
                    @pl.when(c > 0)
                    def _():
                        out_copy(c - 1, half).wait()

                    transpose(s, half)
                    out_copy(c, half).start()
            return carry

        lax.fori_loop(0, _NCH // 2, body, 0)
        out_copy(_NCH - 1, 0).wait()
        out_copy(_NCH - 1, 1).wait()

    return k


_gather = _make_gather()


def kernel(x, emb_weight):
    out5 = _gather(x.astype(jnp.int32), emb_weight)
    return out5.transpose(2, 4, 0, 1, 3).reshape(_BATCH, _HIST, _D)


# speedup vs baseline: 2.2851x; 1.0112x over previous
import functools

import jax
import jax.numpy as jnp
from jax import lax
from jax.experimental import pallas as pl
from jax.experimental.pallas import tpu as pltpu
from jax.experimental.pallas import tpu_sc as plsc

_VOCAB = 1_000_000
_BATCH, _HIST, _D = 16384, 200, 16
_NC, _NS = 2, 16
_NW = _NC * _NS

_GB = 8
_WB = _BATCH // _NW
_NCH = _WB // _GB
_PITCH = 9
_HH = _HIST // 2


def _make_gather():
    mesh = plsc.VectorSubcoreMesh(core_axis_name="c", subcore_axis_name="s")

    @functools.partial(
        pl.kernel,
        mesh=mesh,
        compiler_params=pltpu.CompilerParams(use_tc_tiling_on_sc=False,
                                            needs_layout_passes=False),
        out_type=jax.ShapeDtypeStruct((_HIST, 2, 128, 8, 128), jnp.float32),
        scratch_types=[
            pltpu.VMEM((_GB, _HIST), jnp.int32),
            pltpu.VMEM((_GB, _HIST), jnp.int32),
            pltpu.VMEM((_GB, _HIST, _D), jnp.float32),
            pltpu.VMEM((_GB, _HIST, _D), jnp.float32),
            pltpu.VMEM((_HH, 2, 1, 8, _PITCH), jnp.float32),
            pltpu.VMEM((_HH, 2, 1, 8, _PITCH), jnp.float32),
            pltpu.SemaphoreType.DMA,
            pltpu.SemaphoreType.DMA,
            pltpu.SemaphoreType.DMA,
            pltpu.SemaphoreType.DMA,
            pltpu.SemaphoreType.DMA,
            pltpu.SemaphoreType.DMA,
        ],
    )
    def k(x, table, out, xv0, xv1, rv0, rv1, yv0, yv1,
          i0, i1, g0, g1, o0, o1):
        xv = (xv0, xv1)
        rv = (rv0, rv1)
        yv = (yv0, yv1)
        isem = (i0, i1)
        gsem = (g0, g1)
        osem = (o0, o1)
        wid = lax.axis_index("s") * _NC + lax.axis_index("c")
        wb0 = wid * _WB

        lanes = lax.iota(jnp.int32, 16)
        td_i = lanes >> 3
        tb_i = jnp.zeros((16,), jnp.int32)
        sub_i = lanes & 7

        def idx_copy(c, s):
            return pltpu.make_async_copy(
                x.at[pl.ds(wb0 + c * _GB, _GB), :], xv[s], isem[s])

        def gat_copy(s, g):
            return pltpu.make_async_copy(
                table.at[xv[s].at[g]], rv[s].at[g], gsem[s])

        def out_copy(c, half):
            tb = (wb0 + c * _GB) >> 7
            l0 = pl.multiple_of((wb0 + c * _GB) & 127, _GB)
            return pltpu.make_async_copy(
                yv[half].at[:, :, :, :, pl.ds(0, _GB)],
                out.at[pl.ds(half * _HH, _HH), :, pl.ds(tb, 1), :,
                       pl.ds(l0, _GB)], osem[half])

        def transpose(s, half):
            def hrow(h, carry):
                h_i = jnp.full((16,), h, jnp.int32)
                for g in range(_GB):
                    v = rv[s][g, half * _HH + h, :]
                    plsc.store_scatter(
                        yv[half], [h_i, td_i, tb_i, sub_i,
                                   jnp.full((16,), g, jnp.int32)], v)
                return carry

            lax.fori_loop(0, _HH, hrow, 0, unroll=10)

        idx_copy(0, 0).start()
        idx_copy(1, 1).start()
        idx_copy(0, 0).wait()
        for g in range(_GB):
            gat_copy(0, g).start()

        def body(go, carry):
            for s in range(2):
                c = go * 2 + s
                ns = 1 - s

                @pl.when(c + 1 < _NCH)
                def _():
                    idx_copy(c + 1, ns).wait()
                    for g in range(_GB):
                        gat_copy(ns, g).start()

                for g in range(_GB):
                    gat_copy(s, g).wait()

                @pl.when(c + 2 < _NCH)
                def _():
                    idx_copy(c + 2, s).start()

                for half in range(2):
                    @pl.when(c > 0)
                    def _():
                        out_copy(c - 1, half).wait()

                    transpose(s, half)
                    out_copy(c, half).start()
            return carry

        lax.fori_loop(0, _NCH // 2, body, 0)
        out_copy(_NCH - 1, 0).wait()
        out_copy(_NCH - 1, 1).wait()

    return k


_gather = _make_gather()


def kernel(x, emb_weight):
    out5 = _gather(x.astype(jnp.int32), emb_weight)
    return out5.transpose(2, 4, 0, 1, 3).reshape(_BATCH, _HIST, _D)
